# baseline (speedup 1.0000x reference)
import jax
import jax.numpy as jnp
from jax.experimental import pallas as pl
from jax.experimental.pallas import tpu as pltpu

B = 8
P = 2048
Q = 2048
DPAD = 8
LANE = 128
NBLK = Q // LANE
BPS = 4
NSTEP = B // BPS


def _one_batch(x, y):
    x2 = jnp.sum(x * x, axis=1, keepdims=True)
    aug_x = jnp.concatenate(
        [x, x2, jnp.ones((P, 1), jnp.float32),
         jnp.zeros((P, DPAD - 5), jnp.float32)], axis=1)

    y2 = jnp.sum(y * y, axis=1, keepdims=True)
    aug_y = jnp.concatenate(
        [-2.0 * y, jnp.ones((Q, 1), jnp.float32), y2,
         jnp.zeros((Q, DPAD - 5), jnp.float32)], axis=1)

    d = jax.lax.dot_general(
        aug_x, aug_y, (((1,), (1,)), ((), ())),
        preferred_element_type=jnp.float32,
    )

    s = 0.0
    a = None
    for k in range(NBLK):
        dblk = d[:, k * LANE:(k + 1) * LANE]
        a = dblk if a is None else jnp.minimum(a, dblk)
        my = jnp.min(dblk, axis=0)
        s += jnp.sum(jnp.maximum(my, 0.0))

    mx = jnp.min(a, axis=1)
    return s + jnp.sum(jnp.maximum(mx, 0.0))


def _chamfer_body(x_ref, y_ref, out_ref, acc_ref):
    g = pl.program_id(0)

    s = 0.0
    for i in range(BPS):
        s += _one_batch(x_ref[i], y_ref[i])

    @pl.when(g == 0)
    def _():
        acc_ref[0, 0] = 0.0

    acc_ref[0, 0] += s

    @pl.when(g == NSTEP - 1)
    def _():
        out_ref[0, 0] = acc_ref[0, 0] / (float(P) * float(B))


def kernel(x_hat, pos):
    out = pl.pallas_call(
        _chamfer_body,
        grid=(NSTEP,),
        in_specs=[
            pl.BlockSpec((BPS, P, 3), lambda g: (g, 0, 0)),
            pl.BlockSpec((BPS, Q, 3), lambda g: (g, 0, 0)),
        ],
        out_specs=pl.BlockSpec(
            (1, 1), lambda g: (0, 0), memory_space=pltpu.SMEM
        ),
        out_shape=jax.ShapeDtypeStruct((1, 1), jnp.float32),
        scratch_shapes=[
            pltpu.SMEM((1, 1), jnp.float32),
        ],
    )(x_hat, pos)
    return out[0, 0]


# speedup vs baseline: 1.0357x; 1.0357x over previous
import jax
import jax.numpy as jnp
from jax.experimental import pallas as pl
from jax.experimental.pallas import tpu as pltpu

B = 8
P = 2048
Q = 2048
DPAD = 8
LANE = 128
NBLK = Q // LANE
BPS = 4
NSTEP = B // BPS


def _one_batch(x, y):
    x2 = jnp.sum(x * x, axis=1, keepdims=True)
    aug_x = jnp.concatenate(
        [x, x2, jnp.ones((P, 1), jnp.float32),
         jnp.zeros((P, DPAD - 5), jnp.float32)], axis=1)

    yt = jnp.transpose(y, (1, 0))
    y2 = jnp.sum(yt * yt, axis=0, keepdims=True)
    aug_y = jnp.concatenate(
        [-2.0 * yt, jnp.ones((1, Q), jnp.float32), y2,
         jnp.zeros((DPAD - 5, Q), jnp.float32)], axis=0)

    d = jax.lax.dot_general(
        aug_x, aug_y, (((1,), (0,)), ((), ())),
        preferred_element_type=jnp.float32,
    )

    s = 0.0
    a = None
    for k in range(NBLK):
        dblk = d[:, k * LANE:(k + 1) * LANE]
        a = dblk if a is None else jnp.minimum(a, dblk)
        my = jnp.min(dblk, axis=0)
        s += jnp.sum(jnp.maximum(my, 0.0))

    mx = jnp.min(a, axis=1)
    return s + jnp.sum(jnp.maximum(mx, 0.0))


def _chamfer_body(x_ref, y_ref, out_ref, acc_ref):
    g = pl.program_id(0)

    s = 0.0
    for i in range(BPS):
        s += _one_batch(x_ref[i], y_ref[i])

    @pl.when(g == 0)
    def _():
        acc_ref[0, 0] = 0.0

    acc_ref[0, 0] += s

    @pl.when(g == NSTEP - 1)
    def _():
        out_ref[0, 0] = acc_ref[0, 0] / (float(P) * float(B))


def kernel(x_hat, pos):
    out = pl.pallas_call(
        _chamfer_body,
        grid=(NSTEP,),
        in_specs=[
            pl.BlockSpec((BPS, P, 3), lambda g: (g, 0, 0)),
            pl.BlockSpec((BPS, Q, 3), lambda g: (g, 0, 0)),
        ],
        out_specs=pl.BlockSpec(
            (1, 1), lambda g: (0, 0), memory_space=pltpu.SMEM
        ),
        out_shape=jax.ShapeDtypeStruct((1, 1), jnp.float32),
        scratch_shapes=[
            pltpu.SMEM((1, 1), jnp.float32),
        ],
    )(x_hat, pos)
    return out[0, 0]


# speedup vs baseline: 1.2161x; 1.1742x over previous
import jax
import jax.numpy as jnp
from jax.experimental import pallas as pl
from jax.experimental.pallas import tpu as pltpu

B = 8
P = 2048
Q = 2048
DPAD = 8
LANE = 128
NBLK = Q // LANE
BPS = 4
NSTEP = B // BPS


def _one_batch(x, yt):
    x2 = jnp.sum(x * x, axis=1, keepdims=True)
    aug_x = jnp.concatenate(
        [x, x2, jnp.ones((P, 1), jnp.float32),
         jnp.zeros((P, DPAD - 5), jnp.float32)], axis=1)

    y2 = jnp.sum(yt * yt, axis=0, keepdims=True)
    aug_y = jnp.concatenate(
        [-2.0 * yt, jnp.ones((1, Q), jnp.float32), y2,
         jnp.zeros((DPAD - 5, Q), jnp.float32)], axis=0)

    d = jax.lax.dot_general(
        aug_x, aug_y, (((1,), (0,)), ((), ())),
        preferred_element_type=jnp.float32,
    )

    s = 0.0
    a = None
    for k in range(NBLK):
        dblk = d[:, k * LANE:(k + 1) * LANE]
        a = dblk if a is None else jnp.minimum(a, dblk)
        my = jnp.min(dblk, axis=0)
        s += jnp.sum(jnp.maximum(my, 0.0))

    mx = jnp.min(a, axis=1)
    return s + jnp.sum(jnp.maximum(mx, 0.0))


def _chamfer_body(x_ref, yt_ref, out_ref, acc_ref):
    g = pl.program_id(0)

    s = 0.0
    for i in range(BPS):
        s += _one_batch(x_ref[i], yt_ref[i])

    @pl.when(g == 0)
    def _():
        acc_ref[0, 0] = 0.0

    acc_ref[0, 0] += s

    @pl.when(g == NSTEP - 1)
    def _():
        out_ref[0, 0] = acc_ref[0, 0] / (float(P) * float(B))


def kernel(x_hat, pos):
    ytp = jnp.transpose(pos, (0, 2, 1))

    out = pl.pallas_call(
        _chamfer_body,
        grid=(NSTEP,),
        in_specs=[
            pl.BlockSpec((BPS, P, 3), lambda g: (g, 0, 0)),
            pl.BlockSpec((BPS, 3, Q), lambda g: (g, 0, 0)),
        ],
        out_specs=pl.BlockSpec(
            (1, 1), lambda g: (0, 0), memory_space=pltpu.SMEM
        ),
        out_shape=jax.ShapeDtypeStruct((1, 1), jnp.float32),
        scratch_shapes=[
            pltpu.SMEM((1, 1), jnp.float32),
        ],
        compiler_params=pltpu.CompilerParams(
            allow_input_fusion=[False, True],
        ),
    )(x_hat, ytp)
    return out[0, 0]
